# baseline (speedup 1.0000x reference)
import functools

import jax
import jax.numpy as jnp
from jax import lax
from jax.experimental import pallas as pl
from jax.experimental.pallas import tpu as pltpu
from jax.experimental.pallas import tpu_sc as plsc

B = 16384
CONT = 13
NF = 26
VOCAB = 100000
D = 16

_NC = 2
_NS = 16
_NW = _NC * _NS
_NTOT = B * NF
_PW = _NTOT // _NW
_IPD = 128
_ROWS_PW = _PW // _IPD
_JJ = 8
_CH = _JJ * _IPD
_NCH = _ROWS_PW // _JJ


def _sc_gather_body(idx2d, shared_flat, first_flat, out_rows, out_first,
                    idx_v, rows_v, f_v, sem_r, sem_f):
    c = lax.axis_index("c")
    s = lax.axis_index("s")
    wid = s * _NC + c
    row_base = wid * _ROWS_PW

    def body(ch, carry):
        row0 = row_base + ch * _JJ
        pltpu.sync_copy(idx2d.at[pl.ds(row0, _JJ)], idx_v)
        handles = []
        for j in range(_JJ):
            handles.append(pltpu.async_copy(
                shared_flat.at[idx_v.at[j]],
                rows_v.at[pl.ds(j * _IPD, _IPD)], sem_r))
            handles.append(pltpu.async_copy(
                first_flat.at[idx_v.at[j]],
                f_v.at[pl.ds(j * _IPD, _IPD)], sem_f))
        for h in handles:
            h.wait()
        off = pl.multiple_of(row0 * _IPD, _CH)
        pltpu.sync_copy(rows_v, out_rows.at[pl.ds(off, _CH)])
        pltpu.sync_copy(f_v, out_first.at[pl.ds(off, _CH)])
        return carry

    lax.fori_loop(0, _NCH, body, 0)


def _sc_gather(idx2d, shared_flat, first_flat):
    mesh = plsc.VectorSubcoreMesh(core_axis_name="c", subcore_axis_name="s",
                                  num_cores=_NC, num_subcores=_NS)
    fn = pl.kernel(
        _sc_gather_body,
        out_type=[
            jax.ShapeDtypeStruct((_NTOT, D), jnp.float32),
            jax.ShapeDtypeStruct((_NTOT,), jnp.float32),
        ],
        mesh=mesh,
        scratch_types=[
            pltpu.VMEM((_JJ, _IPD), jnp.int32),
            pltpu.VMEM((_CH, D), jnp.float32),
            pltpu.VMEM((_CH,), jnp.float32),
            pltpu.SemaphoreType.DMA,
            pltpu.SemaphoreType.DMA,
        ],
        compiler_params=pltpu.CompilerParams(use_tc_tiling_on_sc=False),
    )
    return fn(idx2d, shared_flat, first_flat)


def _fm_first_tc_body(c_ref, f_ref, w_ref, b_ref, o_ref):
    cont = c_ref[...]
    w = w_ref[...].reshape(1, CONT)
    lin = jnp.sum(cont * w, axis=1, keepdims=True) + b_ref[0, 0]
    o_ref[...] = jnp.concatenate([lin, f_ref[...]], axis=1)


def _fm_first_tc(cont, firsts, W, b):
    nb = 2048
    return pl.pallas_call(
        _fm_first_tc_body,
        grid=(B // nb,),
        in_specs=[
            pl.BlockSpec((nb, CONT), lambda i: (i, 0)),
            pl.BlockSpec((nb, NF), lambda i: (i, 0)),
            pl.BlockSpec((CONT, 1), lambda i: (0, 0)),
            pl.BlockSpec((1, 1), lambda i: (0, 0)),
        ],
        out_specs=pl.BlockSpec((nb, 1 + NF), lambda i: (i, 0)),
        out_shape=jax.ShapeDtypeStruct((B, 1 + NF), jnp.float32),
    )(cont, firsts, W, b.reshape(1, 1))


def kernel(continuous, categorical, shared_tables, first_tables, W_cont, b_cont):
    idx = categorical + (jnp.arange(NF, dtype=jnp.int32) * VOCAB)[None, :]
    idx2d = idx.reshape(_NTOT // _IPD, _IPD)
    shared_flat = shared_tables.reshape(NF * VOCAB, D)
    first_flat = first_tables.reshape(NF * VOCAB)

    rows, firsts = _sc_gather(idx2d, shared_flat, first_flat)

    fm_second = rows.reshape(B, NF, D)
    dnn = jnp.concatenate([continuous, rows.reshape(B, NF * D)], axis=-1)
    fm_first = _fm_first_tc(continuous, firsts.reshape(B, NF), W_cont, b_cont)
    return dnn, fm_first, fm_second


# speedup vs baseline: 6.5824x; 6.5824x over previous
import jax
import jax.numpy as jnp
from jax import lax
from jax.experimental import pallas as pl
from jax.experimental.pallas import tpu as pltpu
from jax.experimental.pallas import tpu_sc as plsc

B = 16384
CONT = 13
NF = 26
VOCAB = 100000
D = 16

_NC = 2
_NS = 16
_Q = 4096
_NQ = B // _Q
_DNN_W = CONT + NF * D


def _gather_quarters(line_v, idx_v, out_bufs, sem_w, write_targets):
    zero16 = jnp.zeros((16,), jnp.int32)
    handles = {}
    for q in range(_NQ):
        ob = out_bufs[q % 2]
        if q >= 2:
            for h in handles[q - 2]:
                h.wait()

        def gbody(i, carry, _q=q, _ob=ob):
            base = _q * _Q + i * 64
            for u in range(4):
                g = plsc.load_gather(
                    line_v, [zero16, idx_v[0, pl.ds(base + u * 16, 16)]])
                _ob[0, pl.ds(i * 64 + u * 16, 16)] = g
            return carry

        lax.fori_loop(0, _Q // 64, gbody, 0)
        handles[q] = [pltpu.async_copy(ob, dst, sem_w)
                      for dst in write_targets(q * _Q, _Q)]
    for q in (_NQ - 2, _NQ - 1):
        for h in handles[q]:
            h.wait()


def _sc_body(shared_t, cat_t, first2d, cont_t, wb,
             dnn_t, fm1_t, fm2_t,
             line_v, idx_v, out0_v, out1_v, w_v,
             sem_l, sem_w):
    c = lax.axis_index("c")
    s = lax.axis_index("s")
    wid = s * _NC + c
    out_bufs = (out0_v, out1_v)

    @pl.when(wid < NF)
    def _heavy():
        f = wid
        pltpu.sync_copy(cat_t.at[pl.ds(f, 1), pl.ds(0, B)], idx_v)

        def dbody(d, carry):
            pltpu.async_copy(
                shared_t.at[f, pl.ds(d, 1), pl.ds(0, VOCAB)], line_v,
                sem_l).wait()
            r = CONT + f * D + d

            def targets(qo, n):
                return [dnn_t.at[pl.ds(r, 1), pl.ds(qo, n)],
                        fm2_t.at[f, pl.ds(d, 1), pl.ds(qo, n)]]

            _gather_quarters(line_v, idx_v, out_bufs, sem_w, targets)
            return carry

        lax.fori_loop(0, D, dbody, 0)

    @pl.when(wid >= NF)
    def _light():
        lw = wid - NF
        for j in range(5):
            fw = lw + 6 * j

            @pl.when(fw < NF)
            def _first(fw=fw):
                pltpu.sync_copy(cat_t.at[pl.ds(fw, 1), pl.ds(0, B)], idx_v)
                pltpu.async_copy(
                    first2d.at[pl.ds(fw, 1), pl.ds(0, VOCAB)], line_v,
                    sem_l).wait()

                def targets(qo, n):
                    return [fm1_t.at[pl.ds(1 + fw, 1), pl.ds(qo, n)]]

                _gather_quarters(line_v, idx_v, out_bufs, sem_w, targets)

        for cc in range(CONT):
            @pl.when(lw == cc % 6)
            def _cont(cc=cc):
                dst = line_v.at[:, pl.ds(0, B)]
                pltpu.sync_copy(cont_t.at[pl.ds(cc, 1), pl.ds(0, B)], dst)
                pltpu.sync_copy(dst, dnn_t.at[pl.ds(cc, 1), pl.ds(0, B)])

        @pl.when(lw == 5)
        def _lin():
            pltpu.sync_copy(wb, w_v)
            wvec = w_v[0, pl.ds(0, 16)]
            acc = line_v.at[:, pl.ds(0, B)]
            row = line_v.at[:, pl.ds(B, B)]
            for j in range(CONT):
                pltpu.sync_copy(cont_t.at[pl.ds(j, 1), pl.ds(0, B)], row)
                wj = wvec[j]
                if j == 0:
                    bb = wvec[CONT]

                    def ibody(i, carry):
                        acc[0, pl.ds(i * 16, 16)] = (
                            row[0, pl.ds(i * 16, 16)] * wj + bb)
                        return carry

                    lax.fori_loop(0, B // 16, ibody, 0)
                else:
                    def jbody(i, carry, _wj=wj):
                        acc[0, pl.ds(i * 16, 16)] = (
                            acc[0, pl.ds(i * 16, 16)]
                            + row[0, pl.ds(i * 16, 16)] * _wj)
                        return carry

                    lax.fori_loop(0, B // 16, jbody, 0)
            pltpu.sync_copy(acc, fm1_t.at[pl.ds(0, 1), pl.ds(0, B)])


def _sc_call(shared_t, cat_t, first2d, cont_t, wb):
    mesh = plsc.VectorSubcoreMesh(core_axis_name="c", subcore_axis_name="s",
                                  num_cores=_NC, num_subcores=_NS)
    fn = pl.kernel(
        _sc_body,
        out_type=[
            jax.ShapeDtypeStruct((_DNN_W, B), jnp.float32),
            jax.ShapeDtypeStruct((1 + NF, B), jnp.float32),
            jax.ShapeDtypeStruct((NF, D, B), jnp.float32),
        ],
        mesh=mesh,
        scratch_types=[
            pltpu.VMEM((1, VOCAB), jnp.float32),
            pltpu.VMEM((1, B), jnp.int32),
            pltpu.VMEM((1, _Q), jnp.float32),
            pltpu.VMEM((1, _Q), jnp.float32),
            pltpu.VMEM((1, 128), jnp.float32),
            pltpu.SemaphoreType.DMA,
            pltpu.SemaphoreType.DMA,
        ],
        compiler_params=pltpu.CompilerParams(use_tc_tiling_on_sc=True,
                                             needs_layout_passes=False),
    )
    return fn(shared_t, cat_t, first2d, cont_t, wb)


def kernel(continuous, categorical, shared_tables, first_tables, W_cont, b_cont):
    shared_t = jnp.transpose(shared_tables, (0, 2, 1))
    first2d = jnp.transpose(first_tables, (0, 2, 1)).reshape(NF, VOCAB)
    cat_t = categorical.T
    cont_t = continuous.T
    wb = jnp.concatenate(
        [W_cont[:, 0], b_cont,
         jnp.zeros((128 - CONT - 1,), jnp.float32)]).reshape(1, 128)

    dnn_t, fm1_t, fm2_t = _sc_call(shared_t, cat_t, first2d, cont_t, wb)

    dnn = dnn_t.T
    fm_first = fm1_t.T
    fm_second = jnp.transpose(fm2_t, (2, 0, 1))
    return dnn, fm_first, fm_second
